# baseline (speedup 1.0000x reference)
import jax
import jax.numpy as jnp
from jax import lax
from jax.experimental import pallas as pl
from jax.experimental.pallas import tpu as pltpu
from jax.experimental.pallas import tpu_sc as plsc

_NO_CAT = 100000
_EMBED_DIM = 64
_BATCH = 4096
_FIELDS = 26

_B = _BATCH * _FIELDS
_NC = 2
_NS = 16
_NW = _NC * _NS
_B_PER_W = _B // _NW
_CHUNK = 416
_N_CHUNKS = _B_PER_W // _CHUNK


def _gather_kernel(table_hbm, idx_hbm, out_hbm, idx_v, rows_v, sem):
    wid = lax.axis_index("s") * _NC + lax.axis_index("c")
    base = wid * _B_PER_W
    pltpu.sync_copy(idx_hbm.at[pl.ds(base, _B_PER_W)], idx_v)

    for chunk in range(_N_CHUNKS):
        pltpu.async_copy(
            table_hbm.at[idx_v.at[pl.ds(chunk * _CHUNK, _CHUNK)]],
            rows_v,
            sem,
        ).wait()
        pltpu.sync_copy(
            rows_v,
            out_hbm.at[pl.ds(base + chunk * _CHUNK, _CHUNK)],
        )


@jax.jit
def _embedding_lookup(idx_flat, table):
    mesh = plsc.VectorSubcoreMesh(core_axis_name="c", subcore_axis_name="s")
    run = pl.kernel(
        _gather_kernel,
        out_type=jax.ShapeDtypeStruct((_B, _EMBED_DIM), jnp.float32),
        mesh=mesh,
        scratch_types=[
            pltpu.VMEM((_B_PER_W,), jnp.int32),
            pltpu.VMEM((_CHUNK, _EMBED_DIM), jnp.float32),
            pltpu.SemaphoreType.DMA,
        ],
        compiler_params=pltpu.CompilerParams(use_tc_tiling_on_sc=False),
    )
    return run(table, idx_flat)


def kernel(x, table):
    idx_flat = x.reshape(_B).astype(jnp.int32)
    out = _embedding_lookup(idx_flat, table)
    return out.reshape(_BATCH, _FIELDS, _EMBED_DIM)


# speedup vs baseline: 1.0276x; 1.0276x over previous
import jax
import jax.numpy as jnp
from jax import lax
from jax.experimental import pallas as pl
from jax.experimental.pallas import tpu as pltpu
from jax.experimental.pallas import tpu_sc as plsc

_NO_CAT = 100000
_EMBED_DIM = 64
_BATCH = 4096
_FIELDS = 26

_B = _BATCH * _FIELDS
_NC = 2
_NS = 16
_NW = _NC * _NS
_B_PER_W = _B // _NW
_CHUNK = 416
_N_CHUNKS = _B_PER_W // _CHUNK
_NBUF = 4


def _gather_kernel(table_hbm, idx_hbm, out_hbm, idx_v, rows_v, *sems):
    gsems, osems = sems[:_NBUF], sems[_NBUF:]
    wid = lax.axis_index("s") * _NC + lax.axis_index("c")
    base = wid * _B_PER_W
    pltpu.sync_copy(idx_hbm.at[pl.ds(base, _B_PER_W)], idx_v)

    def g_desc(chunk, b):
        return pltpu.make_async_copy(
            table_hbm.at[idx_v.at[pl.ds(chunk * _CHUNK, _CHUNK)]],
            rows_v.at[b],
            gsems[b],
        )

    def o_desc(chunk, b):
        return pltpu.make_async_copy(
            rows_v.at[b],
            out_hbm.at[pl.ds(base + chunk * _CHUNK, _CHUNK)],
            osems[b],
        )

    for b in range(_NBUF):
        g_desc(b, b).start()

    for chunk in range(_N_CHUNKS):
        b = chunk % _NBUF
        g_desc(chunk, b).wait()
        o_desc(chunk, b).start()
        prev = chunk - 1
        if prev >= 0 and prev + _NBUF < _N_CHUNKS:
            o_desc(prev, prev % _NBUF).wait()
            g_desc(prev + _NBUF, prev % _NBUF).start()

    for chunk in range(_N_CHUNKS - _NBUF, _N_CHUNKS):
        o_desc(chunk, chunk % _NBUF).wait()


@jax.jit
def _embedding_lookup(idx_flat, table):
    mesh = plsc.VectorSubcoreMesh(core_axis_name="c", subcore_axis_name="s")
    run = pl.kernel(
        _gather_kernel,
        out_type=jax.ShapeDtypeStruct((_B, _EMBED_DIM), jnp.float32),
        mesh=mesh,
        scratch_types=[
            pltpu.VMEM((_B_PER_W,), jnp.int32),
            pltpu.VMEM((_NBUF, _CHUNK, _EMBED_DIM), jnp.float32),
        ] + [pltpu.SemaphoreType.DMA] * (2 * _NBUF),
        compiler_params=pltpu.CompilerParams(use_tc_tiling_on_sc=False),
    )
    return run(table, idx_flat)


def kernel(x, table):
    idx_flat = x.reshape(_B).astype(jnp.int32)
    out = _embedding_lookup(idx_flat, table)
    return out.reshape(_BATCH, _FIELDS, _EMBED_DIM)
